# Initial kernel scaffold; baseline (speedup 1.0000x reference)
import jax
import jax.numpy as jnp
from jax.experimental import pallas as pl


def kernel(cncpt_ids, edge_index, concept_table, W1, b1, W2, b2, Wout, bout):
    raise NotImplementedError("write your pallas kernel here")


# speedup vs baseline: 5.1112x; 5.1112x over previous
import functools

import jax
import jax.numpy as jnp
from jax import lax
from jax.experimental import pallas as pl
from jax.experimental.pallas import tpu as pltpu
from jax.experimental.pallas import tpu_sc as plsc

N_NODES_C = 10000
N_PAD = 10240
N_EDGES_C = 320000
D = 128

NC = 2
NS = 16
NW = NC * NS
E_PER_W = N_EDGES_C // NW
CHUNK = 80
N_CHUNKS = E_PER_W // CHUNK
ROWS_PER_TILE = N_PAD // NS


def _make_edge_layer(compose: bool):
  mesh = plsc.VectorSubcoreMesh(core_axis_name="c", subcore_axis_name="s")

  @functools.partial(
      pl.kernel,
      mesh=mesh,
      compiler_params=pltpu.CompilerParams(needs_layout_passes=False),
      out_type=jax.ShapeDtypeStruct((NC, N_PAD, D), jnp.float32),
      scratch_types=[
          pltpu.VMEM((CHUNK,), jnp.int32),
          pltpu.VMEM((CHUNK,), jnp.int32),
          pltpu.VMEM((CHUNK,), jnp.int32),
          pltpu.VMEM((CHUNK, D), jnp.float32),
          pltpu.VMEM((N_NODES_C,), jnp.int32),
          pltpu.VMEM_SHARED((N_PAD, D), jnp.float32),
          pltpu.SemaphoreType.DMA,
      ],
  )
  def edge_layer(ids_hbm, src_hbm, dst_hbm, table_hbm, zeros_hbm, out_hbm,
                 src_v, dst_v, eff_v, rows_v, ids_v, acc, sem):
    c = lax.axis_index("c")
    s = lax.axis_index("s")
    wid = c * NS + s

    row0 = s * ROWS_PER_TILE
    pltpu.sync_copy(zeros_hbm.at[pl.ds(row0, ROWS_PER_TILE)],
                    acc.at[pl.ds(row0, ROWS_PER_TILE)])
    if compose:
      pltpu.sync_copy(ids_hbm, ids_v)
    plsc.subcore_barrier()

    def body(k, carry):
      off = wid * E_PER_W + k * CHUNK
      pltpu.sync_copy(src_hbm.at[pl.ds(off, CHUNK)], src_v)
      pltpu.sync_copy(dst_hbm.at[pl.ds(off, CHUNK)], dst_v)
      if compose:
        for j in range(CHUNK // 16):
          idx = src_v[pl.ds(j * 16, 16)]
          eff_v[pl.ds(j * 16, 16)] = plsc.load_gather(ids_v, [idx])
        gather_idx = eff_v
      else:
        gather_idx = src_v
      pltpu.async_copy(table_hbm.at[gather_idx], rows_v, sem).wait()
      pltpu.sync_copy(rows_v, acc.at[dst_v], add=True)
      return carry

    lax.fori_loop(0, N_CHUNKS, body, 0)
    plsc.subcore_barrier()

    pltpu.sync_copy(acc.at[pl.ds(row0, ROWS_PER_TILE)],
                    out_hbm.at[c, pl.ds(row0, ROWS_PER_TILE)])

  return edge_layer


_edge_layer1 = _make_edge_layer(compose=True)
_edge_layer2 = _make_edge_layer(compose=False)


def _dense_relu_kernel(p_ref, w_ref, b_ref, o_ref):
  h = p_ref[0] + p_ref[1]
  o_ref[...] = jax.nn.relu(
      jnp.dot(h, w_ref[...], preferred_element_type=jnp.float32) + b_ref[...])


def _dense_head_kernel(p_ref, w_ref, b_ref, wo_ref, bo_ref, o_ref):
  h = p_ref[0] + p_ref[1]
  h2 = jax.nn.relu(
      jnp.dot(h, w_ref[...], preferred_element_type=jnp.float32) + b_ref[...])
  o_ref[...] = jax.nn.sigmoid(
      jnp.dot(h2, wo_ref[...], preferred_element_type=jnp.float32)
      + bo_ref[...])


_ROWS_BLK = 2048


def _dense_relu(partials, w, b):
  return pl.pallas_call(
      _dense_relu_kernel,
      grid=(N_PAD // _ROWS_BLK,),
      in_specs=[
          pl.BlockSpec((NC, _ROWS_BLK, D), lambda i: (0, i, 0)),
          pl.BlockSpec((D, D), lambda i: (0, 0)),
          pl.BlockSpec((1, D), lambda i: (0, 0)),
      ],
      out_specs=pl.BlockSpec((_ROWS_BLK, D), lambda i: (i, 0)),
      out_shape=jax.ShapeDtypeStruct((N_PAD, D), jnp.float32),
  )(partials, w, b.reshape(1, D))


def _dense_head(partials, w, b, wout, bout):
  return pl.pallas_call(
      _dense_head_kernel,
      grid=(N_PAD // _ROWS_BLK,),
      in_specs=[
          pl.BlockSpec((NC, _ROWS_BLK, D), lambda i: (0, i, 0)),
          pl.BlockSpec((D, D), lambda i: (0, 0)),
          pl.BlockSpec((1, D), lambda i: (0, 0)),
          pl.BlockSpec((D, 1), lambda i: (0, 0)),
          pl.BlockSpec((1, 1), lambda i: (0, 0)),
      ],
      out_specs=pl.BlockSpec((_ROWS_BLK, 1), lambda i: (i, 0)),
      out_shape=jax.ShapeDtypeStruct((N_PAD, 1), jnp.float32),
  )(partials, w, b.reshape(1, D), wout, bout.reshape(1, 1))


@jax.jit
def kernel(cncpt_ids, edge_index, concept_table, W1, b1, W2, b2, Wout, bout):
  src = edge_index[0]
  dst = edge_index[1]
  zeros = jnp.zeros((N_PAD, D), jnp.float32)

  p1 = _edge_layer1(cncpt_ids, src, dst, concept_table, zeros)
  h1 = _dense_relu(p1, W1, b1)
  p2 = _edge_layer2(cncpt_ids, src, dst, h1, zeros)
  logits = _dense_head(p2, W2, b2, Wout, bout)
  return logits[None, :N_NODES_C, :]
